# baseline (speedup 1.0000x reference)
import functools

import jax
import jax.numpy as jnp
from jax import lax
from jax.experimental import pallas as pl
from jax.experimental.pallas import tpu as pltpu
from jax.experimental.pallas import tpu_sc as plsc

N = 10000
E = 320000
D = 128
DH = D // 2
DOUT = 768

NC = 2
NS = 16
NW = NC * NS
CH = 125
ROWS2 = E // CH
NCH = ROWS2 // NS
NCHD = ROWS2 // NW
NPAD = 10240
TR = NPAD // NS
ZB = 128

_mesh = plsc.VectorSubcoreMesh(core_axis_name="c", subcore_axis_name="s")


@functools.partial(
    pl.kernel,
    out_type=jax.ShapeDtypeStruct((NC, NPAD), jnp.float32),
    mesh=_mesh,
    scratch_types=[
        pltpu.VMEM((NCHD, CH), jnp.int32),
        pltpu.VMEM((128,), jnp.float32),
        pltpu.VMEM((TR,), jnp.float32),
        pltpu.VMEM_SHARED((NPAD,), jnp.float32),
    ],
)
def _sc_deg(dst_hbm, out_hbm, dstb, ones_v, zer_v, sh_deg):
    cid = lax.axis_index("c")
    sid = lax.axis_index("s")
    wid = sid * NC + cid

    def _fill(i, _):
        zer_v[pl.ds(i * 16, 16)] = jnp.zeros((16,), jnp.float32)
        return 0
    lax.fori_loop(0, TR // 16, _fill, 0)

    def _fill1(i, _):
        ones_v[pl.ds(i * 16, 16)] = jnp.full((16,), 1.0, jnp.float32)
        return 0
    lax.fori_loop(0, 8, _fill1, 0)

    pltpu.sync_copy(zer_v, sh_deg.at[pl.ds(sid * TR, TR)])
    pltpu.sync_copy(dst_hbm.at[pl.ds(wid * NCHD, NCHD)], dstb)
    plsc.subcore_barrier()

    def _acc(j, _):
        pltpu.sync_copy(ones_v.at[pl.ds(0, CH)], sh_deg.at[dstb.at[j]],
                        add=True)
        return 0
    lax.fori_loop(0, NCHD, _acc, 0)
    plsc.subcore_barrier()

    pltpu.sync_copy(sh_deg.at[pl.ds(sid * TR, TR)],
                    out_hbm.at[cid, pl.ds(sid * TR, TR)])


@functools.partial(
    pl.kernel,
    out_type=jax.ShapeDtypeStruct((NC, NPAD, DH), jnp.float32),
    mesh=_mesh,
    scratch_types=[
        pltpu.VMEM((NCH, CH), jnp.int32),
        pltpu.VMEM((NCH, CH), jnp.int32),
        pltpu.VMEM((CH, DH), jnp.float32),
        pltpu.VMEM((CH, DH), jnp.float32),
        pltpu.VMEM((CH, DH), jnp.float32),
        pltpu.VMEM((CH, DH), jnp.float32),
        pltpu.VMEM((ZB, DH), jnp.float32),
        pltpu.SemaphoreType.DMA,
        pltpu.SemaphoreType.DMA,
        pltpu.SemaphoreType.DMA,
        pltpu.SemaphoreType.DMA,
        pltpu.SemaphoreType.DMA,
        pltpu.SemaphoreType.DMA,
        pltpu.SemaphoreType.DMA,
        pltpu.SemaphoreType.DMA,
        pltpu.VMEM_SHARED((NPAD, DH), jnp.float32),
    ],
    compiler_params=pltpu.CompilerParams(use_tc_tiling_on_sc=False),
)
def _sc_agg(x0_hbm, x1_hbm, src_hbm, dst_hbm, out_hbm,
            srcb, dstb, rows0, rows1, rows2, rows3, zbuf,
            g0, g1, g2, g3, s0, s1, s2, s3, sh_agg):
    cid = lax.axis_index("c")
    sid = lax.axis_index("s")
    rows = [rows0, rows1, rows2, rows3]
    gsem = [g0, g1, g2, g3]
    ssem = [s0, s1, s2, s3]

    def _zero(i, _):
        for j in range(DH // 16):
            zbuf[i, pl.ds(j * 16, 16)] = jnp.zeros((16,), jnp.float32)
        return 0
    lax.fori_loop(0, ZB, _zero, 0)
    for k in range(TR // ZB):
        pltpu.sync_copy(zbuf, sh_agg.at[pl.ds(sid * TR + k * ZB, ZB)])

    pltpu.sync_copy(src_hbm.at[pl.ds(sid * NCH, NCH)], srcb)
    pltpu.sync_copy(dst_hbm.at[pl.ds(sid * NCH, NCH)], dstb)
    plsc.subcore_barrier()

    def _run(x_hbm):
        def _gath(t, b):
            pltpu.async_copy(x_hbm.at[srcb.at[t]], rows[b], gsem[b])

        def _gath_wait(t, b):
            pltpu.make_async_copy(x_hbm.at[srcb.at[t]], rows[b],
                                  gsem[b]).wait()

        def _scat(c, b):
            pltpu.async_copy(rows[b], sh_agg.at[dstb.at[c]], ssem[b],
                             add=True)

        def _scat_wait(c, b):
            pltpu.make_async_copy(rows[b], sh_agg.at[dstb.at[c]],
                                  ssem[b]).wait()

        _gath(0, 0)
        _gath(1, 1)
        _gath(2, 2)
        _gath_wait(0, 0)
        _scat(0, 0)
        _gath(3, 3)
        _gath_wait(1, 1)
        _scat(1, 1)

        def _step(jo, _):
            for b in range(4):
                t = jo * 4 + b
                _scat_wait(t - 4, b)
                _gath(t, b)
                bs = (b + 2) % 4
                _gath_wait(t - 2, bs)
                _scat(t - 2, bs)
            return 0
        lax.fori_loop(1, NCH // 4, _step, 0)

        _gath_wait(NCH - 2, (NCH - 2) % 4)
        _scat(NCH - 2, (NCH - 2) % 4)
        _gath_wait(NCH - 1, (NCH - 1) % 4)
        _scat(NCH - 1, (NCH - 1) % 4)
        for b in range(4):
            _scat_wait(NCH - 4 + b, b)

    @pl.when(cid == 0)
    def _():
        _run(x0_hbm)

    @pl.when(cid == 1)
    def _():
        _run(x1_hbm)

    plsc.subcore_barrier()
    pltpu.sync_copy(sh_agg.at[pl.ds(sid * TR, TR)],
                    out_hbm.at[cid, pl.ds(sid * TR, TR)])


_BR = 1000


def _tc1(emb, W1, d0, d1):
    def body(e_ref, w_ref, d0_ref, d1_ref, oL_ref, oR_ref):
        s = lax.rsqrt(jnp.maximum(d0_ref[...] + d1_ref[...], 1.0))
        x = jnp.dot(e_ref[...], w_ref[...],
                    preferred_element_type=jnp.float32) * s
        oL_ref[...] = x[:, :DH]
        oR_ref[...] = x[:, DH:]
    return pl.pallas_call(
        body,
        grid=(N // _BR,),
        in_specs=[
            pl.BlockSpec((_BR, D), lambda i: (i, 0)),
            pl.BlockSpec((D, D), lambda i: (0, 0)),
            pl.BlockSpec((_BR, 1), lambda i: (i, 0)),
            pl.BlockSpec((_BR, 1), lambda i: (i, 0)),
        ],
        out_specs=[pl.BlockSpec((_BR, DH), lambda i: (i, 0)),
                   pl.BlockSpec((_BR, DH), lambda i: (i, 0))],
        out_shape=[jax.ShapeDtypeStruct((N, DH), jnp.float32),
                   jax.ShapeDtypeStruct((N, DH), jnp.float32)],
    )(emb, W1, d0, d1)


def _tc2(ag, d0, d1, b1L, b1R, W2T, W2B):
    def body(aL_ref, aR_ref, d0_ref, d1_ref, bL_ref, bR_ref, wT_ref, wB_ref,
             oL_ref, oR_ref):
        s = lax.rsqrt(jnp.maximum(d0_ref[...] + d1_ref[...], 1.0))
        hL = jnp.maximum(s * aL_ref[0] + bL_ref[...], 0.0)
        hR = jnp.maximum(s * aR_ref[0] + bR_ref[...], 0.0)
        o = (jnp.dot(hL, wT_ref[...], preferred_element_type=jnp.float32)
             + jnp.dot(hR, wB_ref[...], preferred_element_type=jnp.float32))
        o = o * s
        oL_ref[...] = o[:, :DH]
        oR_ref[...] = o[:, DH:]
    return pl.pallas_call(
        body,
        grid=(N // _BR,),
        in_specs=[
            pl.BlockSpec((1, _BR, DH), lambda i: (0, i, 0)),
            pl.BlockSpec((1, _BR, DH), lambda i: (1, i, 0)),
            pl.BlockSpec((_BR, 1), lambda i: (i, 0)),
            pl.BlockSpec((_BR, 1), lambda i: (i, 0)),
            pl.BlockSpec((1, DH), lambda i: (0, 0)),
            pl.BlockSpec((1, DH), lambda i: (0, 0)),
            pl.BlockSpec((DH, D), lambda i: (0, 0)),
            pl.BlockSpec((DH, D), lambda i: (0, 0)),
        ],
        out_specs=[pl.BlockSpec((_BR, DH), lambda i: (i, 0)),
                   pl.BlockSpec((_BR, DH), lambda i: (i, 0))],
        out_shape=[jax.ShapeDtypeStruct((N, DH), jnp.float32),
                   jax.ShapeDtypeStruct((N, DH), jnp.float32)],
    )(ag, ag, d0, d1, b1L, b1R, W2T, W2B)


def _tc3(ag, d0, d1, b2L, b2R, Wp1T, Wp1B, bp1, Wp2, bp2):
    def body(aL_ref, aR_ref, d0_ref, d1_ref, bL_ref, bR_ref, wT_ref, wB_ref,
             bp1_ref, w2_ref, bp2_ref, o_ref):
        s = lax.rsqrt(jnp.maximum(d0_ref[...] + d1_ref[...], 1.0))
        hL = s * aL_ref[0] + bL_ref[...]
        hR = s * aR_ref[0] + bR_ref[...]
        p = jnp.maximum(
            jnp.dot(hL, wT_ref[...], preferred_element_type=jnp.float32)
            + jnp.dot(hR, wB_ref[...], preferred_element_type=jnp.float32)
            + bp1_ref[...], 0.0)
        o = jnp.dot(p, w2_ref[...],
                    preferred_element_type=jnp.float32) + bp2_ref[...]
        nrm = jnp.sqrt(jnp.sum(o * o, axis=1, keepdims=True))
        o_ref[...] = o / jnp.maximum(nrm, 1e-12)
    return pl.pallas_call(
        body,
        grid=(N // _BR,),
        in_specs=[
            pl.BlockSpec((1, _BR, DH), lambda i: (0, i, 0)),
            pl.BlockSpec((1, _BR, DH), lambda i: (1, i, 0)),
            pl.BlockSpec((_BR, 1), lambda i: (i, 0)),
            pl.BlockSpec((_BR, 1), lambda i: (i, 0)),
            pl.BlockSpec((1, DH), lambda i: (0, 0)),
            pl.BlockSpec((1, DH), lambda i: (0, 0)),
            pl.BlockSpec((DH, D), lambda i: (0, 0)),
            pl.BlockSpec((DH, D), lambda i: (0, 0)),
            pl.BlockSpec((1, D), lambda i: (0, 0)),
            pl.BlockSpec((D, DOUT), lambda i: (0, 0)),
            pl.BlockSpec((1, DOUT), lambda i: (0, 0)),
        ],
        out_specs=pl.BlockSpec((_BR, DOUT), lambda i: (i, 0)),
        out_shape=jax.ShapeDtypeStruct((N, DOUT), jnp.float32),
    )(ag, ag, d0, d1, b2L, b2R, Wp1T, Wp1B, bp1, Wp2, bp2)


def kernel(adj, emb, W1, b1, W2, b2, Wp1, bp1, Wp2, bp2):
    src2 = adj[0].reshape(ROWS2, CH)
    dst2 = adj[1].reshape(ROWS2, CH)

    degp = _sc_deg(dst2)
    d0 = degp[0, :N].reshape(N, 1)
    d1 = degp[1, :N].reshape(N, 1)

    x1L, x1R = _tc1(emb, W1, d0, d1)
    ag1 = _sc_agg(x1L, x1R, src2, dst2)
    x2L, x2R = _tc2(ag1, d0, d1,
                    b1[:DH].reshape(1, DH), b1[DH:].reshape(1, DH),
                    W2[:DH], W2[DH:])
    ag2 = _sc_agg(x2L, x2R, src2, dst2)
    return _tc3(ag2, d0, d1,
                b2[:DH].reshape(1, DH), b2[DH:].reshape(1, DH),
                Wp1[:DH], Wp1[DH:], bp1.reshape(1, D),
                Wp2, bp2.reshape(1, DOUT))


# speedup vs baseline: 1.1591x; 1.1591x over previous
import functools

import jax
import jax.numpy as jnp
from jax import lax
from jax.experimental import pallas as pl
from jax.experimental.pallas import tpu as pltpu
from jax.experimental.pallas import tpu_sc as plsc

N = 10000
E = 320000
D = 128
DH = D // 2
DOUT = 768

NC = 2
NS = 16
NW = NC * NS
CH = 125
ROWS2 = E // CH
NCH = ROWS2 // NS
NCHD = ROWS2 // NW
NPAD = 10240
TR = NPAD // NS
ZB = 128

_mesh = plsc.VectorSubcoreMesh(core_axis_name="c", subcore_axis_name="s")


@functools.partial(
    pl.kernel,
    out_type=jax.ShapeDtypeStruct((NC, NPAD), jnp.float32),
    mesh=_mesh,
    scratch_types=[
        pltpu.VMEM((NCHD, CH), jnp.int32),
        pltpu.VMEM((128,), jnp.float32),
        pltpu.VMEM((TR,), jnp.float32),
        pltpu.VMEM_SHARED((NPAD,), jnp.float32),
    ],
)
def _sc_deg(dst_hbm, out_hbm, dstb, ones_v, zer_v, sh_deg):
    cid = lax.axis_index("c")
    sid = lax.axis_index("s")
    wid = sid * NC + cid

    def _fill(i, _):
        zer_v[pl.ds(i * 16, 16)] = jnp.zeros((16,), jnp.float32)
        return 0
    lax.fori_loop(0, TR // 16, _fill, 0)

    def _fill1(i, _):
        ones_v[pl.ds(i * 16, 16)] = jnp.full((16,), 1.0, jnp.float32)
        return 0
    lax.fori_loop(0, 8, _fill1, 0)

    pltpu.sync_copy(zer_v, sh_deg.at[pl.ds(sid * TR, TR)])
    pltpu.sync_copy(dst_hbm.at[pl.ds(wid * NCHD, NCHD)], dstb)
    plsc.subcore_barrier()

    def _acc(j, _):
        pltpu.sync_copy(ones_v.at[pl.ds(0, CH)], sh_deg.at[dstb.at[j]],
                        add=True)
        return 0
    lax.fori_loop(0, NCHD, _acc, 0)
    plsc.subcore_barrier()

    pltpu.sync_copy(sh_deg.at[pl.ds(sid * TR, TR)],
                    out_hbm.at[cid, pl.ds(sid * TR, TR)])


@functools.partial(
    pl.kernel,
    out_type=jax.ShapeDtypeStruct((NPAD, D), jnp.float32),
    mesh=_mesh,
    scratch_types=[
        pltpu.VMEM((NCH, CH), jnp.int32),
        pltpu.VMEM((NCH, CH), jnp.int32),
        pltpu.VMEM((CH, DH), jnp.float32),
        pltpu.VMEM((CH, DH), jnp.float32),
        pltpu.VMEM((CH, DH), jnp.float32),
        pltpu.VMEM((CH, DH), jnp.float32),
        pltpu.VMEM((ZB, DH), jnp.float32),
        pltpu.SemaphoreType.DMA,
        pltpu.SemaphoreType.DMA,
        pltpu.SemaphoreType.DMA,
        pltpu.SemaphoreType.DMA,
        pltpu.SemaphoreType.DMA,
        pltpu.SemaphoreType.DMA,
        pltpu.SemaphoreType.DMA,
        pltpu.SemaphoreType.DMA,
        pltpu.VMEM_SHARED((NPAD, DH), jnp.float32),
    ],
    compiler_params=pltpu.CompilerParams(use_tc_tiling_on_sc=False),
)
def _sc_agg(x_hbm, srcA_hbm, srcB_hbm, dst_hbm, out_hbm,
            srcb, dstb, rows0, rows1, rows2, rows3, zbuf,
            g0, g1, g2, g3, s0, s1, s2, s3, sh_agg):
    cid = lax.axis_index("c")
    sid = lax.axis_index("s")
    rows = [rows0, rows1, rows2, rows3]
    gsem = [g0, g1, g2, g3]
    ssem = [s0, s1, s2, s3]

    def _zero(i, _):
        for j in range(DH // 16):
            zbuf[i, pl.ds(j * 16, 16)] = jnp.zeros((16,), jnp.float32)
        return 0
    lax.fori_loop(0, ZB, _zero, 0)
    for k in range(TR // ZB):
        pltpu.sync_copy(zbuf, sh_agg.at[pl.ds(sid * TR + k * ZB, ZB)])

    @pl.when(cid == 0)
    def _():
        pltpu.sync_copy(srcA_hbm.at[pl.ds(sid * NCH, NCH)], srcb)

    @pl.when(cid == 1)
    def _():
        pltpu.sync_copy(srcB_hbm.at[pl.ds(sid * NCH, NCH)], srcb)

    pltpu.sync_copy(dst_hbm.at[pl.ds(sid * NCH, NCH)], dstb)
    plsc.subcore_barrier()

    def _gath(t, b):
        pltpu.async_copy(x_hbm.at[srcb.at[t]], rows[b], gsem[b])

    def _gath_wait(t, b):
        pltpu.make_async_copy(x_hbm.at[srcb.at[t]], rows[b],
                              gsem[b]).wait()

    def _scat(c, b):
        pltpu.async_copy(rows[b], sh_agg.at[dstb.at[c]], ssem[b],
                         add=True)

    def _scat_wait(c, b):
        pltpu.make_async_copy(rows[b], sh_agg.at[dstb.at[c]],
                              ssem[b]).wait()

    _gath(0, 0)
    _gath(1, 1)
    _gath(2, 2)
    _gath_wait(0, 0)
    _scat(0, 0)
    _gath(3, 3)
    _gath_wait(1, 1)
    _scat(1, 1)

    def _step(jo, _):
        for b in range(4):
            t = jo * 4 + b
            _scat_wait(t - 4, b)
            _gath(t, b)
            bs = (b + 2) % 4
            _gath_wait(t - 2, bs)
            _scat(t - 2, bs)
        return 0
    lax.fori_loop(1, NCH // 4, _step, 0)

    _gath_wait(NCH - 2, (NCH - 2) % 4)
    _scat(NCH - 2, (NCH - 2) % 4)
    _gath_wait(NCH - 1, (NCH - 1) % 4)
    _scat(NCH - 1, (NCH - 1) % 4)
    for b in range(4):
        _scat_wait(NCH - 4 + b, b)

    plsc.subcore_barrier()
    pltpu.sync_copy(sh_agg.at[pl.ds(sid * TR, TR)],
                    out_hbm.at[pl.ds(sid * TR, TR), pl.ds(cid * DH, DH)])


_BR = 1000


def _tc1(emb, W1, s):
    def body(e_ref, w_ref, s_ref, o_ref):
        o_ref[...] = jnp.dot(e_ref[...], w_ref[...],
                             preferred_element_type=jnp.float32) * s_ref[...]
    return pl.pallas_call(
        body,
        grid=(N // _BR,),
        in_specs=[
            pl.BlockSpec((_BR, D), lambda i: (i, 0)),
            pl.BlockSpec((D, D), lambda i: (0, 0)),
            pl.BlockSpec((_BR, 1), lambda i: (i, 0)),
        ],
        out_specs=pl.BlockSpec((_BR, D), lambda i: (i, 0)),
        out_shape=jax.ShapeDtypeStruct((N, D), jnp.float32),
    )(emb, W1, s)


def _tc2(ag, s, b1, W2):
    def body(a_ref, s_ref, b_ref, w_ref, o_ref):
        h = jnp.maximum(s_ref[...] * a_ref[...] + b_ref[...], 0.0)
        o_ref[...] = jnp.dot(h, w_ref[...],
                             preferred_element_type=jnp.float32) * s_ref[...]
    return pl.pallas_call(
        body,
        grid=(N // _BR,),
        in_specs=[
            pl.BlockSpec((_BR, D), lambda i: (i, 0)),
            pl.BlockSpec((_BR, 1), lambda i: (i, 0)),
            pl.BlockSpec((1, D), lambda i: (0, 0)),
            pl.BlockSpec((D, D), lambda i: (0, 0)),
        ],
        out_specs=pl.BlockSpec((_BR, D), lambda i: (i, 0)),
        out_shape=jax.ShapeDtypeStruct((N, D), jnp.float32),
    )(ag, s, b1, W2)


def _tc3(ag, s, b2, Wp1, bp1, Wp2, bp2):
    def body(a_ref, s_ref, b_ref, w1_ref, bp1_ref, w2_ref, bp2_ref, o_ref):
        h = s_ref[...] * a_ref[...] + b_ref[...]
        p = jnp.maximum(
            jnp.dot(h, w1_ref[...], preferred_element_type=jnp.float32)
            + bp1_ref[...], 0.0)
        o = jnp.dot(p, w2_ref[...],
                    preferred_element_type=jnp.float32) + bp2_ref[...]
        nrm = jnp.sqrt(jnp.sum(o * o, axis=1, keepdims=True))
        o_ref[...] = o / jnp.maximum(nrm, 1e-12)
    return pl.pallas_call(
        body,
        grid=(N // _BR,),
        in_specs=[
            pl.BlockSpec((_BR, D), lambda i: (i, 0)),
            pl.BlockSpec((_BR, 1), lambda i: (i, 0)),
            pl.BlockSpec((1, D), lambda i: (0, 0)),
            pl.BlockSpec((D, D), lambda i: (0, 0)),
            pl.BlockSpec((1, D), lambda i: (0, 0)),
            pl.BlockSpec((D, DOUT), lambda i: (0, 0)),
            pl.BlockSpec((1, DOUT), lambda i: (0, 0)),
        ],
        out_specs=pl.BlockSpec((_BR, DOUT), lambda i: (i, 0)),
        out_shape=jax.ShapeDtypeStruct((N, DOUT), jnp.float32),
    )(ag, s, b2, Wp1, bp1, Wp2, bp2)


def kernel(adj, emb, W1, b1, W2, b2, Wp1, bp1, Wp2, bp2):
    src2 = adj[0].reshape(ROWS2, CH)
    dst2 = adj[1].reshape(ROWS2, CH)
    srcA = src2 * 2
    srcB = srcA + 1

    degp = _sc_deg(dst2)
    s = lax.rsqrt(jnp.maximum(degp[0, :N] + degp[1, :N], 1.0)).reshape(N, 1)

    x1 = _tc1(emb, W1, s)
    ag1 = _sc_agg(x1.reshape(2 * N, DH), srcA, srcB, dst2)
    x2 = _tc2(ag1, s, b1.reshape(1, D), W2)
    ag2 = _sc_agg(x2.reshape(2 * N, DH), srcA, srcB, dst2)
    return _tc3(ag2, s, b2.reshape(1, D),
                Wp1, bp1.reshape(1, D), Wp2, bp2.reshape(1, DOUT))
